# baseline (speedup 1.0000x reference)
import functools

import jax
import jax.numpy as jnp
import numpy as np
from jax import lax
from jax.experimental import pallas as pl
from jax.experimental.pallas import tpu as pltpu
from jax.experimental.pallas import tpu_sc as plsc

N_NODES = 10000
N_EDGES = 320000
D_FEAT = 128
N_HEADS = 8
D_HEAD = 16

NC = 2
NS = 16
NW = NC * NS
EPT = N_EDGES // NW
BLK = 80
NG = BLK // 16
NBLK = EPT // BLK
RPT = 624
TAIL = N_NODES - NS * RPT

_PERM = np.empty((D_FEAT,), np.int32)
for _c in range(D_FEAT // 32):
    for _i in range(16):
        _PERM[32 * _c + 2 * _i] = 32 * _c + _i
        _PERM[32 * _c + 2 * _i + 1] = 32 * _c + 16 + _i


def _proj_body(x_ref, bq_ref, bkv_ref, q_ref, kv_ref):
    xb = x_ref[...]
    q_ref[...] = jnp.dot(xb, bq_ref[...], preferred_element_type=jnp.float32)
    kv_ref[...] = jnp.dot(
        xb, bkv_ref[...], preferred_element_type=jnp.float32
    ).astype(jnp.bfloat16)


def _project(x, bq, bkv):
    nb = 10
    rows = N_NODES // nb
    return pl.pallas_call(
        _proj_body,
        grid=(nb,),
        in_specs=[
            pl.BlockSpec((rows, D_FEAT), lambda i: (i, 0)),
            pl.BlockSpec((D_FEAT, D_FEAT), lambda i: (0, 0)),
            pl.BlockSpec((D_FEAT, 2 * D_FEAT), lambda i: (0, 0)),
        ],
        out_specs=[
            pl.BlockSpec((rows, D_FEAT), lambda i: (i, 0)),
            pl.BlockSpec((rows, 2 * D_FEAT), lambda i: (i, 0)),
        ],
        out_shape=[
            jax.ShapeDtypeStruct((N_NODES, D_FEAT), jnp.float32),
            jax.ShapeDtypeStruct((N_NODES, 2 * D_FEAT), jnp.bfloat16),
        ],
    )(x, bq, bkv)


def _scale_body(p_ref, m_ref, o_ref):
    o_ref[...] = p_ref[...] * m_ref[...] * 0.25


def _edge_scale(phi, msk):
    rows = N_EDGES // 256
    out = pl.pallas_call(
        _scale_body,
        grid=(1,),
        in_specs=[
            pl.BlockSpec((rows, 256), lambda i: (0, 0)),
            pl.BlockSpec((rows, 256), lambda i: (0, 0)),
        ],
        out_specs=pl.BlockSpec((rows, 256), lambda i: (0, 0)),
        out_shape=jax.ShapeDtypeStruct((rows, 256), jnp.float32),
    )(phi.reshape(rows, 256), msk.reshape(rows, 256))
    return out.reshape(N_EDGES)


def _combine_body(p_ref, o_ref):
    o_ref[...] = p_ref[0] + p_ref[1]


def _combine(partials):
    nb = 10
    rows = N_NODES // nb
    return pl.pallas_call(
        _combine_body,
        grid=(nb,),
        in_specs=[pl.BlockSpec((2, rows, D_FEAT), lambda i: (0, i, 0))],
        out_specs=pl.BlockSpec((rows, D_FEAT), lambda i: (i, 0)),
        out_shape=jax.ShapeDtypeStruct((N_NODES, D_FEAT), jnp.float32),
    )(partials)


_MESH = plsc.VectorSubcoreMesh(core_axis_name="c", subcore_axis_name="s")


@functools.partial(
    pl.kernel,
    out_type=jax.ShapeDtypeStruct((NC, N_NODES, D_FEAT), jnp.float32),
    mesh=_MESH,
    scratch_types=[
        pltpu.VMEM((3, BLK), jnp.int32),
        pltpu.VMEM((BLK, D_FEAT), jnp.float32),
        pltpu.VMEM((BLK, D_FEAT), jnp.float32),
        pltpu.VMEM((BLK, D_FEAT), jnp.int32),
        pltpu.VMEM((BLK, D_FEAT), jnp.float32),
        pltpu.VMEM_SHARED((N_NODES, D_FEAT), jnp.float32),
        pltpu.SemaphoreType.DMA,
        pltpu.SemaphoreType.DMA,
        pltpu.SemaphoreType.DMA,
    ],
    compiler_params=pltpu.CompilerParams(needs_layout_passes=False),
)
def _sc_edges(q_hbm, kv_hbm, w_hbm, meta_hbm,
              zero_hbm, out_hbm,
              meta_v, w_v, q_v, kv_v, msg_v, part,
              semL, semQ, semK):
    cid = lax.axis_index("c")
    sid = lax.axis_index("s")
    wid = cid * NS + sid

    pltpu.sync_copy(zero_hbm.at[pl.ds(sid * RPT, RPT)],
                    part.at[pl.ds(sid * RPT, RPT)])

    @pl.when(sid == 0)
    def _zero_tail():
        pltpu.sync_copy(zero_hbm.at[pl.ds(NS * RPT, TAIL)],
                        part.at[pl.ds(NS * RPT, TAIL)])

    plsc.subcore_barrier()

    def block(b, carry):
        base = wid * EPT + b * BLK
        lin = [
            pltpu.async_copy(meta_hbm.at[wid * NBLK + b], meta_v, semL),
            pltpu.async_copy(w_hbm.at[pl.ds(base, BLK)], w_v, semL),
        ]
        for cp in lin:
            cp.wait()
        gq = pltpu.async_copy(q_hbm.at[meta_v.at[0]], q_v, semQ)
        gk = pltpu.async_copy(kv_hbm.at[meta_v.at[1]], kv_v, semK)
        gq.wait()
        gk.wait()

        iota = lax.iota(jnp.int32, 16)

        def group(g, carry2):
            sv = plsc.bitcast(meta_v[2, pl.ds(g * 16, 16)], jnp.float32)

            @plsc.parallel_loop(0, 16, 1, unroll=4)
            def _edge(j):
                e = g * 16 + j
                scv = jnp.take(sv, iota * 0 + j)
                for c in range(D_FEAT // 32):
                    kk = plsc.bitcast(kv_v[e, pl.ds(16 * c, 16)],
                                      jnp.bfloat16)
                    vv = plsc.bitcast(
                        kv_v[e, pl.ds(D_FEAT // 2 + 16 * c, 16)],
                        jnp.bfloat16)
                    ka, kb = plsc.unpack(
                        kk, format=plsc.PackFormat.INTERLEAVED)
                    va, vb = plsc.unpack(
                        vv, format=plsc.PackFormat.INTERLEAVED)
                    for hh, kx, vx in ((2 * c, ka, va),
                                       (2 * c + 1, kb, vb)):
                        qx = q_v[e, pl.ds(hh * D_HEAD, D_HEAD)]
                        wh = w_v[e, pl.ds(hh * D_HEAD, D_HEAD)]
                        p = qx * wh * kx
                        for sh in (8, 4, 2, 1):
                            p = p + jnp.take(p, (iota + sh) & 15)
                        msg_v[e, pl.ds(hh * D_HEAD, D_HEAD)] = vx * (p * scv)

            return carry2

        lax.fori_loop(0, NG, group, 0)
        pltpu.sync_copy(msg_v, part.at[meta_v.at[0]], add=True)
        return carry

    lax.fori_loop(0, NBLK, block, 0)
    plsc.subcore_barrier()
    pltpu.sync_copy(part.at[pl.ds(sid * RPT, RPT)],
                    out_hbm.at[cid, pl.ds(sid * RPT, RPT)])

    @pl.when(sid == 0)
    def _write_tail():
        pltpu.sync_copy(part.at[pl.ds(NS * RPT, TAIL)],
                        out_hbm.at[cid, pl.ds(NS * RPT, TAIL)])


def kernel(x, w_ij, phi_r_cut, idx_i, idx_j, pair_mask, Wq, Wk, Wv):
    eye = jnp.eye(N_HEADS, dtype=jnp.float32)
    bq = jnp.einsum('hfg,hk->hfkg', Wq, eye).reshape(D_FEAT, D_FEAT)
    bk = jnp.einsum('hfg,hk->hfkg', Wk, eye).reshape(D_FEAT, D_FEAT)
    bv = jnp.einsum('hfg,hk->hfkg', Wv, eye).reshape(D_FEAT, D_FEAT)
    perm = jnp.asarray(_PERM)
    bkv = jnp.concatenate([bk[:, perm], bv[:, perm]], axis=1)

    q, kv = _project(x, bq, bkv)
    kv = lax.bitcast_convert_type(
        kv.reshape(N_NODES, D_FEAT, 2), jnp.int32)
    scale = _edge_scale(phi_r_cut, pair_mask)
    meta = jnp.stack(
        [idx_i.astype(jnp.int32).reshape(NW * NBLK, BLK),
         idx_j.astype(jnp.int32).reshape(NW * NBLK, BLK),
         lax.bitcast_convert_type(scale.reshape(NW * NBLK, BLK), jnp.int32)],
        axis=1)
    zeros = jnp.zeros((N_NODES, D_FEAT), jnp.float32)
    partials = _sc_edges(q, kv, w_ij, meta, zeros)
    return _combine(partials)


# speedup vs baseline: 1.2294x; 1.2294x over previous
import functools

import jax
import jax.numpy as jnp
import numpy as np
from jax import lax
from jax.experimental import pallas as pl
from jax.experimental.pallas import tpu as pltpu
from jax.experimental.pallas import tpu_sc as plsc

N_NODES = 10000
N_EDGES = 320000
D_FEAT = 128
N_HEADS = 8
D_HEAD = 16

NC = 2
NS = 16
NW = NC * NS
BLK = 64
NG = BLK // 16
NBT = N_EDGES // BLK
RPT = 624
TAIL = N_NODES - NS * RPT

_PERM = np.empty((D_FEAT,), np.int32)
for _c in range(D_FEAT // 32):
    for _i in range(16):
        _PERM[32 * _c + 2 * _i] = 32 * _c + _i
        _PERM[32 * _c + 2 * _i + 1] = 32 * _c + 16 + _i


def _proj_body(x_ref, bq_ref, bkv_ref, q_ref, kv_ref):
    xb = x_ref[...]
    q_ref[...] = jnp.dot(xb, bq_ref[...], preferred_element_type=jnp.float32)
    kv_ref[...] = jnp.dot(
        xb, bkv_ref[...], preferred_element_type=jnp.float32
    ).astype(jnp.bfloat16)


def _project(x, bq, bkv):
    nb = 10
    rows = N_NODES // nb
    return pl.pallas_call(
        _proj_body,
        grid=(nb,),
        in_specs=[
            pl.BlockSpec((rows, D_FEAT), lambda i: (i, 0)),
            pl.BlockSpec((D_FEAT, D_FEAT), lambda i: (0, 0)),
            pl.BlockSpec((D_FEAT, 2 * D_FEAT), lambda i: (0, 0)),
        ],
        out_specs=[
            pl.BlockSpec((rows, D_FEAT), lambda i: (i, 0)),
            pl.BlockSpec((rows, 2 * D_FEAT), lambda i: (i, 0)),
        ],
        out_shape=[
            jax.ShapeDtypeStruct((N_NODES, D_FEAT), jnp.float32),
            jax.ShapeDtypeStruct((N_NODES, 2 * D_FEAT), jnp.bfloat16),
        ],
    )(x, bq, bkv)


def _scale_body(p_ref, m_ref, o_ref):
    o_ref[...] = p_ref[...] * m_ref[...] * 0.25


def _edge_scale(phi, msk):
    rows = N_EDGES // 256
    out = pl.pallas_call(
        _scale_body,
        grid=(1,),
        in_specs=[
            pl.BlockSpec((rows, 256), lambda i: (0, 0)),
            pl.BlockSpec((rows, 256), lambda i: (0, 0)),
        ],
        out_specs=pl.BlockSpec((rows, 256), lambda i: (0, 0)),
        out_shape=jax.ShapeDtypeStruct((rows, 256), jnp.float32),
    )(phi.reshape(rows, 256), msk.reshape(rows, 256))
    return out.reshape(N_EDGES)


def _combine_body(p_ref, o_ref):
    o_ref[...] = p_ref[0] + p_ref[1]


def _combine(partials):
    nb = 10
    rows = N_NODES // nb
    return pl.pallas_call(
        _combine_body,
        grid=(nb,),
        in_specs=[pl.BlockSpec((2, rows, D_FEAT), lambda i: (0, i, 0))],
        out_specs=pl.BlockSpec((rows, D_FEAT), lambda i: (i, 0)),
        out_shape=jax.ShapeDtypeStruct((N_NODES, D_FEAT), jnp.float32),
    )(partials)


_MESH = plsc.VectorSubcoreMesh(core_axis_name="c", subcore_axis_name="s")


@functools.partial(
    pl.kernel,
    out_type=jax.ShapeDtypeStruct((NC, N_NODES, D_FEAT), jnp.float32),
    mesh=_MESH,
    scratch_types=[
        pltpu.VMEM((3, BLK), jnp.int32),
        pltpu.VMEM((3, BLK), jnp.int32),
        pltpu.VMEM((BLK, D_FEAT), jnp.float32),
        pltpu.VMEM((BLK, D_FEAT), jnp.float32),
        pltpu.VMEM((BLK, D_FEAT), jnp.float32),
        pltpu.VMEM((BLK, D_FEAT), jnp.float32),
        pltpu.VMEM((BLK, D_FEAT), jnp.int32),
        pltpu.VMEM((BLK, D_FEAT), jnp.int32),
        pltpu.VMEM_SHARED((N_NODES, D_FEAT), jnp.float32),
        pltpu.SemaphoreType.DMA,
        pltpu.SemaphoreType.DMA,
        pltpu.SemaphoreType.DMA,
        pltpu.SemaphoreType.DMA,
        pltpu.SemaphoreType.DMA,
        pltpu.SemaphoreType.DMA,
        pltpu.SemaphoreType.DMA,
        pltpu.SemaphoreType.DMA,
    ],
    compiler_params=pltpu.CompilerParams(needs_layout_passes=False),
)
def _sc_edges(q_hbm, kv_hbm, w_hbm, meta_hbm,
              zero_hbm, out_hbm,
              meta0, meta1, w0, w1, q0, q1, kv0, kv1, part,
              semL0, semL1, semQ0, semQ1, semK0, semK1, semS0, semS1):
    cid = lax.axis_index("c")
    sid = lax.axis_index("s")
    wid = cid * NS + sid
    start = 156 * wid + 2 * jnp.minimum(wid, 4)
    nsb = 78 + jnp.where(wid < 4, 1, 0)

    pltpu.sync_copy(zero_hbm.at[pl.ds(sid * RPT, RPT)],
                    part.at[pl.ds(sid * RPT, RPT)])

    @pl.when(sid == 0)
    def _zero_tail():
        pltpu.sync_copy(zero_hbm.at[pl.ds(NS * RPT, TAIL)],
                        part.at[pl.ds(NS * RPT, TAIL)])

    plsc.subcore_barrier()

    iota = lax.iota(jnp.int32, 16)

    def compute(meta_v, w_v, q_v, kv_v):
        def group(g, carry2):
            sv = plsc.bitcast(meta_v[2, pl.ds(g * 16, 16)], jnp.float32)

            @plsc.parallel_loop(0, 16, 1, unroll=4)
            def _edge(j):
                e = g * 16 + j
                scv = jnp.take(sv, iota * 0 + j)
                for c in range(D_FEAT // 32):
                    kk = plsc.bitcast(kv_v[e, pl.ds(16 * c, 16)],
                                      jnp.bfloat16)
                    vv = plsc.bitcast(
                        kv_v[e, pl.ds(D_FEAT // 2 + 16 * c, 16)],
                        jnp.bfloat16)
                    ka, kb = plsc.unpack(
                        kk, format=plsc.PackFormat.INTERLEAVED)
                    va, vb = plsc.unpack(
                        vv, format=plsc.PackFormat.INTERLEAVED)
                    for hh, kx, vx in ((2 * c, ka, va),
                                       (2 * c + 1, kb, vb)):
                        qx = q_v[e, pl.ds(hh * D_HEAD, D_HEAD)]
                        wh = w_v[e, pl.ds(hh * D_HEAD, D_HEAD)]
                        p = qx * wh * kx
                        for sh in (8, 4, 2, 1):
                            p = p + jnp.take(p, (iota + sh) & 15)
                        w_v[e, pl.ds(hh * D_HEAD, D_HEAD)] = vx * (p * scv)

            return carry2

        lax.fori_loop(0, NG, group, 0)

    def superblock(sb, carry):
        b0 = start + 2 * sb
        b1 = b0 + 1
        l0m = pltpu.async_copy(meta_hbm.at[b0], meta0, semL0)
        l0w = pltpu.async_copy(w_hbm.at[pl.ds(b0 * BLK, BLK)], w0, semL0)
        l1m = pltpu.async_copy(meta_hbm.at[b1], meta1, semL1)
        l1w = pltpu.async_copy(w_hbm.at[pl.ds(b1 * BLK, BLK)], w1, semL1)
        l0m.wait()
        l0w.wait()
        g0q = pltpu.async_copy(q_hbm.at[meta0.at[0]], q0, semQ0)
        g0k = pltpu.async_copy(kv_hbm.at[meta0.at[1]], kv0, semK0)
        l1m.wait()
        l1w.wait()
        g1q = pltpu.async_copy(q_hbm.at[meta1.at[0]], q1, semQ1)
        g1k = pltpu.async_copy(kv_hbm.at[meta1.at[1]], kv1, semK1)
        g0q.wait()
        g0k.wait()
        compute(meta0, w0, q0, kv0)
        s0 = pltpu.async_copy(w0, part.at[meta0.at[0]], semS0, add=True)
        g1q.wait()
        g1k.wait()
        compute(meta1, w1, q1, kv1)
        s1 = pltpu.async_copy(w1, part.at[meta1.at[0]], semS1, add=True)
        s0.wait()
        s1.wait()
        return carry

    lax.fori_loop(0, nsb, superblock, 0)
    plsc.subcore_barrier()
    pltpu.sync_copy(part.at[pl.ds(sid * RPT, RPT)],
                    out_hbm.at[cid, pl.ds(sid * RPT, RPT)])

    @pl.when(sid == 0)
    def _write_tail():
        pltpu.sync_copy(part.at[pl.ds(NS * RPT, TAIL)],
                        out_hbm.at[cid, pl.ds(NS * RPT, TAIL)])


def kernel(x, w_ij, phi_r_cut, idx_i, idx_j, pair_mask, Wq, Wk, Wv):
    eye = jnp.eye(N_HEADS, dtype=jnp.float32)
    bq = jnp.einsum('hfg,hk->hfkg', Wq, eye).reshape(D_FEAT, D_FEAT)
    bk = jnp.einsum('hfg,hk->hfkg', Wk, eye).reshape(D_FEAT, D_FEAT)
    bv = jnp.einsum('hfg,hk->hfkg', Wv, eye).reshape(D_FEAT, D_FEAT)
    perm = jnp.asarray(_PERM)
    bkv = jnp.concatenate([bk[:, perm], bv[:, perm]], axis=1)

    q, kv = _project(x, bq, bkv)
    kv = lax.bitcast_convert_type(
        kv.reshape(N_NODES, D_FEAT, 2), jnp.int32)
    scale = _edge_scale(phi_r_cut, pair_mask)
    meta = jnp.stack(
        [idx_i.astype(jnp.int32).reshape(NBT, BLK),
         idx_j.astype(jnp.int32).reshape(NBT, BLK),
         lax.bitcast_convert_type(scale.reshape(NBT, BLK), jnp.int32)],
        axis=1)
    zeros = jnp.zeros((N_NODES, D_FEAT), jnp.float32)
    partials = _sc_edges(q, kv, w_ij, meta, zeros)
    return _combine(partials)


# speedup vs baseline: 1.2338x; 1.0036x over previous
import functools

import jax
import jax.numpy as jnp
import numpy as np
from jax import lax
from jax.experimental import pallas as pl
from jax.experimental.pallas import tpu as pltpu
from jax.experimental.pallas import tpu_sc as plsc

N_NODES = 10000
N_EDGES = 320000
D_FEAT = 128
N_HEADS = 8
D_HEAD = 16

NC = 2
NS = 16
NW = NC * NS
BLK = 64
NG = BLK // 16
NBT = N_EDGES // BLK
RPT = 624
TAIL = N_NODES - NS * RPT

_PERM = np.empty((D_FEAT,), np.int32)
for _c in range(D_FEAT // 32):
    for _i in range(16):
        _PERM[32 * _c + 2 * _i] = 32 * _c + _i
        _PERM[32 * _c + 2 * _i + 1] = 32 * _c + 16 + _i


def _proj_body(x_ref, bq_ref, bkv_ref, q_ref, kv_ref):
    xb = x_ref[...]
    q_ref[...] = jnp.dot(xb, bq_ref[...], preferred_element_type=jnp.float32)
    kv_ref[...] = jnp.dot(
        xb, bkv_ref[...], preferred_element_type=jnp.float32
    ).astype(jnp.bfloat16)


def _project(x, bq, bkv):
    nb = 10
    rows = N_NODES // nb
    return pl.pallas_call(
        _proj_body,
        grid=(nb,),
        in_specs=[
            pl.BlockSpec((rows, D_FEAT), lambda i: (i, 0)),
            pl.BlockSpec((D_FEAT, D_FEAT), lambda i: (0, 0)),
            pl.BlockSpec((D_FEAT, 2 * D_FEAT), lambda i: (0, 0)),
        ],
        out_specs=[
            pl.BlockSpec((rows, D_FEAT), lambda i: (i, 0)),
            pl.BlockSpec((rows, 2 * D_FEAT), lambda i: (i, 0)),
        ],
        out_shape=[
            jax.ShapeDtypeStruct((N_NODES, D_FEAT), jnp.float32),
            jax.ShapeDtypeStruct((N_NODES, 2 * D_FEAT), jnp.bfloat16),
        ],
    )(x, bq, bkv)


def _scale_body(p_ref, m_ref, o_ref):
    o_ref[...] = p_ref[...] * m_ref[...] * 0.25


def _edge_scale(phi, msk):
    rows = N_EDGES // 256
    out = pl.pallas_call(
        _scale_body,
        grid=(1,),
        in_specs=[
            pl.BlockSpec((rows, 256), lambda i: (0, 0)),
            pl.BlockSpec((rows, 256), lambda i: (0, 0)),
        ],
        out_specs=pl.BlockSpec((rows, 256), lambda i: (0, 0)),
        out_shape=jax.ShapeDtypeStruct((rows, 256), jnp.float32),
    )(phi.reshape(rows, 256), msk.reshape(rows, 256))
    return out.reshape(N_EDGES)


def _combine_body(p_ref, o_ref):
    o_ref[...] = p_ref[0] + p_ref[1]


def _combine(partials):
    nb = 10
    rows = N_NODES // nb
    return pl.pallas_call(
        _combine_body,
        grid=(nb,),
        in_specs=[pl.BlockSpec((2, rows, D_FEAT), lambda i: (0, i, 0))],
        out_specs=pl.BlockSpec((rows, D_FEAT), lambda i: (i, 0)),
        out_shape=jax.ShapeDtypeStruct((N_NODES, D_FEAT), jnp.float32),
    )(partials)


_MESH = plsc.VectorSubcoreMesh(core_axis_name="c", subcore_axis_name="s")


@functools.partial(
    pl.kernel,
    out_type=jax.ShapeDtypeStruct((NC, N_NODES, D_FEAT), jnp.float32),
    mesh=_MESH,
    scratch_types=[
        pltpu.VMEM((3, BLK), jnp.int32),
        pltpu.VMEM((3, BLK), jnp.int32),
        pltpu.VMEM((BLK, D_FEAT), jnp.float32),
        pltpu.VMEM((BLK, D_FEAT), jnp.float32),
        pltpu.VMEM((BLK, D_FEAT), jnp.float32),
        pltpu.VMEM((BLK, D_FEAT), jnp.float32),
        pltpu.VMEM((BLK, D_FEAT), jnp.int32),
        pltpu.VMEM((BLK, D_FEAT), jnp.int32),
        pltpu.VMEM_SHARED((N_NODES, D_FEAT), jnp.float32),
        pltpu.SemaphoreType.DMA,
        pltpu.SemaphoreType.DMA,
        pltpu.SemaphoreType.DMA,
        pltpu.SemaphoreType.DMA,
        pltpu.SemaphoreType.DMA,
        pltpu.SemaphoreType.DMA,
        pltpu.SemaphoreType.DMA,
        pltpu.SemaphoreType.DMA,
    ],
    compiler_params=pltpu.CompilerParams(needs_layout_passes=False),
)
def _sc_edges(q_hbm, kv_hbm, w_hbm, meta_hbm,
              zero_hbm, out_hbm,
              meta0, meta1, w0, w1, q0, q1, kv0, kv1, part,
              semL0, semL1, semQ0, semQ1, semK0, semK1, semS0, semS1):
    cid = lax.axis_index("c")
    sid = lax.axis_index("s")
    wid = cid * NS + sid
    start = 156 * wid + 2 * jnp.minimum(wid, 4)
    nsb = 78 + jnp.where(wid < 4, 1, 0)

    pltpu.sync_copy(zero_hbm.at[pl.ds(sid * RPT, RPT)],
                    part.at[pl.ds(sid * RPT, RPT)])

    @pl.when(sid == 0)
    def _zero_tail():
        pltpu.sync_copy(zero_hbm.at[pl.ds(NS * RPT, TAIL)],
                        part.at[pl.ds(NS * RPT, TAIL)])

    plsc.subcore_barrier()

    iota = lax.iota(jnp.int32, 16)

    def compute(meta_v, w_v, q_v, kv_v):
        def group(g, carry2):
            sv = plsc.bitcast(meta_v[2, pl.ds(g * 16, 16)], jnp.float32)

            @plsc.parallel_loop(0, 16, 1, unroll=8)
            def _edge(j):
                e = g * 16 + j
                scv = jnp.take(sv, iota * 0 + j)
                for c in range(D_FEAT // 32):
                    kk = plsc.bitcast(kv_v[e, pl.ds(16 * c, 16)],
                                      jnp.bfloat16)
                    vv = plsc.bitcast(
                        kv_v[e, pl.ds(D_FEAT // 2 + 16 * c, 16)],
                        jnp.bfloat16)
                    ka, kb = plsc.unpack(
                        kk, format=plsc.PackFormat.INTERLEAVED)
                    va, vb = plsc.unpack(
                        vv, format=plsc.PackFormat.INTERLEAVED)
                    for hh, kx, vx in ((2 * c, ka, va),
                                       (2 * c + 1, kb, vb)):
                        qx = q_v[e, pl.ds(hh * D_HEAD, D_HEAD)]
                        wh = w_v[e, pl.ds(hh * D_HEAD, D_HEAD)]
                        p = qx * wh * kx
                        for sh in (8, 4, 2, 1):
                            p = p + jnp.take(p, (iota + sh) & 15)
                        w_v[e, pl.ds(hh * D_HEAD, D_HEAD)] = vx * (p * scv)

            return carry2

        lax.fori_loop(0, NG, group, 0)

    def superblock(sb, carry):
        b0 = start + 2 * sb
        b1 = b0 + 1
        l0m = pltpu.async_copy(meta_hbm.at[b0], meta0, semL0)
        l0w = pltpu.async_copy(w_hbm.at[pl.ds(b0 * BLK, BLK)], w0, semL0)
        l1m = pltpu.async_copy(meta_hbm.at[b1], meta1, semL1)
        l1w = pltpu.async_copy(w_hbm.at[pl.ds(b1 * BLK, BLK)], w1, semL1)
        l0m.wait()
        l0w.wait()
        g0q = pltpu.async_copy(q_hbm.at[meta0.at[0]], q0, semQ0)
        g0k = pltpu.async_copy(kv_hbm.at[meta0.at[1]], kv0, semK0)
        l1m.wait()
        l1w.wait()
        g1q = pltpu.async_copy(q_hbm.at[meta1.at[0]], q1, semQ1)
        g1k = pltpu.async_copy(kv_hbm.at[meta1.at[1]], kv1, semK1)
        g0q.wait()
        g0k.wait()
        compute(meta0, w0, q0, kv0)
        s0 = pltpu.async_copy(w0, part.at[meta0.at[0]], semS0, add=True)
        g1q.wait()
        g1k.wait()
        compute(meta1, w1, q1, kv1)
        s1 = pltpu.async_copy(w1, part.at[meta1.at[0]], semS1, add=True)
        s0.wait()
        s1.wait()
        return carry

    lax.fori_loop(0, nsb, superblock, 0)
    plsc.subcore_barrier()
    pltpu.sync_copy(part.at[pl.ds(sid * RPT, RPT)],
                    out_hbm.at[cid, pl.ds(sid * RPT, RPT)])

    @pl.when(sid == 0)
    def _write_tail():
        pltpu.sync_copy(part.at[pl.ds(NS * RPT, TAIL)],
                        out_hbm.at[cid, pl.ds(NS * RPT, TAIL)])


def kernel(x, w_ij, phi_r_cut, idx_i, idx_j, pair_mask, Wq, Wk, Wv):
    eye = jnp.eye(N_HEADS, dtype=jnp.float32)
    bq = jnp.einsum('hfg,hk->hfkg', Wq, eye).reshape(D_FEAT, D_FEAT)
    bk = jnp.einsum('hfg,hk->hfkg', Wk, eye).reshape(D_FEAT, D_FEAT)
    bv = jnp.einsum('hfg,hk->hfkg', Wv, eye).reshape(D_FEAT, D_FEAT)
    perm = jnp.asarray(_PERM)
    bkv = jnp.concatenate([bk[:, perm], bv[:, perm]], axis=1)

    q, kv = _project(x, bq, bkv)
    kv = lax.bitcast_convert_type(
        kv.reshape(N_NODES, D_FEAT, 2), jnp.int32)
    scale = _edge_scale(phi_r_cut, pair_mask)
    meta = jnp.stack(
        [idx_i.astype(jnp.int32).reshape(NBT, BLK),
         idx_j.astype(jnp.int32).reshape(NBT, BLK),
         lax.bitcast_convert_type(scale.reshape(NBT, BLK), jnp.int32)],
        axis=1)
    zeros = jnp.zeros((N_NODES, D_FEAT), jnp.float32)
    partials = _sc_edges(q, kv, w_ij, meta, zeros)
    return _combine(partials)
